# baseline (speedup 1.0000x reference)
import functools

import jax
import jax.numpy as jnp
from jax import lax
from jax.experimental import pallas as pl
from jax.experimental.pallas import tpu as pltpu
from jax.experimental.pallas import tpu_sc as plsc

DESIRED = 5.0
UNDESIRED = 1.0

_NC = 2
_NS = 16
_NW = _NC * _NS


def _sc_body(x_hbm, t_hbm, o_hbm, buf_ref, acc_ref, tbuf_ref, sq_ref,
             stage_ref, sem, *, T, n_tile_rows, scale):
    wid = lax.axis_index("s") * _NC + lax.axis_index("c")

    def start(t, c0, slot):
        pltpu.make_async_copy(
            x_hbm.at[t, pl.ds(c0, 8), :],
            buf_ref.at[slot],
            sem.at[slot],
        ).start()

    def wait(t, c0, slot):
        pltpu.make_async_copy(
            x_hbm.at[t, pl.ds(c0, 8), :],
            buf_ref.at[slot],
            sem.at[slot],
        ).wait()

    sq_ref[...] = jnp.zeros((16,), jnp.float32)

    for task in range(4):
        tr = wid + task * _NW

        @pl.when(tr < n_tile_rows)
        def _():
            c0 = tr * 8

            def zero_row(r, carry):
                for j in range(16):
                    acc_ref[r, pl.ds(j * 16, 16)] = jnp.zeros((16,),
                                                              jnp.float32)
                return carry
            lax.fori_loop(0, 8, zero_row, 0)

            start(0, c0, 0)

            def step(t, carry):
                slot = lax.rem(t, 2)
                wait(t, c0, slot)

                def add_row(r, carry2):
                    for j in range(16):
                        sl = pl.ds(j * 16, 16)
                        acc_ref[r, sl] = acc_ref[r, sl] + buf_ref[slot, r, sl]
                    return carry2
                lax.fori_loop(0, 8, add_row, 0)

                @pl.when(t + 1 < T)
                def _():
                    start(t + 1, c0, 1 - slot)

                return carry
            lax.fori_loop(0, T, step, 0)

            pltpu.sync_copy(t_hbm.at[pl.ds(c0, 8), :], tbuf_ref)

            def sq_row(r, carry):
                for j in range(16):
                    sl = pl.ds(j * 16, 16)
                    oc = acc_ref[r, sl]
                    tg = tbuf_ref[r, sl]
                    des = jnp.full((16,), DESIRED, jnp.float32)
                    und = jnp.full((16,), UNDESIRED, jnp.float32)
                    oc = jnp.where((tg == des) & (oc > des), des, oc)
                    oc = jnp.where((tg == und) & (oc < und), und, oc)
                    d = oc - tg
                    sq_ref[...] = sq_ref[...] + d * d
                return carry
            lax.fori_loop(0, 8, sq_row, 0)

    stage_ref[...] = sq_ref[...] * scale
    pltpu.sync_copy(stage_ref, o_hbm.at[wid])


def kernel(output, target):
    T, N, C = output.shape
    assert C % 8 == 0
    n_tile_rows = C // 8
    scale = 0.5 / T

    xt = jnp.transpose(output, (0, 2, 1))
    tt = target.T

    mesh = plsc.VectorSubcoreMesh(core_axis_name="c", subcore_axis_name="s")
    k = functools.partial(
        pl.kernel,
        out_type=jax.ShapeDtypeStruct((_NW, 16), jnp.float32),
        mesh=mesh,
        scratch_types=[
            pltpu.VMEM((2, 8, N), jnp.float32),
            pltpu.VMEM((8, N), jnp.float32),
            pltpu.VMEM((8, N), jnp.float32),
            pltpu.VMEM((16,), jnp.float32),
            pltpu.VMEM((16,), jnp.float32),
            pltpu.SemaphoreType.DMA((2,)),
        ],
    )(functools.partial(_sc_body, T=T, n_tile_rows=n_tile_rows, scale=scale))

    partials = k(xt, tt)
    return jnp.sum(partials)


# speedup vs baseline: 17.1652x; 17.1652x over previous
import functools

import jax
import jax.numpy as jnp
from jax.experimental import pallas as pl
from jax.experimental.pallas import tpu as pltpu

DESIRED = 5.0
UNDESIRED = 1.0


def _body(x_hbm, t_hbm, o_ref, acc_ref, buf_ref, tbuf_ref, sem, tsem,
          *, K, TB, NBUF, scale):
    def start(j):
        slot = jax.lax.rem(j, NBUF)
        pltpu.make_async_copy(
            x_hbm.at[pl.ds(j * TB, TB)],
            buf_ref.at[slot],
            sem.at[slot],
        ).start()

    def wait(j):
        slot = jax.lax.rem(j, NBUF)
        pltpu.make_async_copy(
            x_hbm.at[pl.ds(j * TB, TB)],
            buf_ref.at[slot],
            sem.at[slot],
        ).wait()

    tcopy = pltpu.make_async_copy(t_hbm, tbuf_ref, tsem)
    tcopy.start()
    for j in range(min(NBUF, K)):
        start(j)

    def step(j, carry):
        wait(j)

        slot = jax.lax.rem(j, NBUF)
        s = jnp.sum(buf_ref[slot], axis=0)

        @pl.when(j == 0)
        def _():
            acc_ref[...] = s

        @pl.when(j > 0)
        def _():
            acc_ref[...] += s

        @pl.when(j + NBUF < K)
        def _():
            start(j + NBUF)

        return carry

    jax.lax.fori_loop(0, K, step, 0, unroll=False)

    tcopy.wait()
    t = tbuf_ref[...]
    oc = acc_ref[...]
    oc = jnp.where((t == DESIRED) & (oc > DESIRED), DESIRED, oc)
    oc = jnp.where((t == UNDESIRED) & (oc < UNDESIRED), UNDESIRED, oc)
    d = oc - t
    o_ref[0, 0] = jnp.sum(d * d) * scale


def kernel(output, target):
    T, N, C = output.shape
    TB = 4
    NBUF = 8
    assert T % TB == 0
    K = T // TB
    scale = 0.5 / T

    xt = jnp.transpose(output, (0, 2, 1))
    tt = target.T

    out = pl.pallas_call(
        functools.partial(_body, K=K, TB=TB, NBUF=NBUF, scale=scale),
        in_specs=[
            pl.BlockSpec(memory_space=pl.ANY),
            pl.BlockSpec(memory_space=pl.ANY),
        ],
        out_specs=pl.BlockSpec(memory_space=pltpu.SMEM),
        out_shape=jax.ShapeDtypeStruct((1, 1), jnp.float32),
        scratch_shapes=[
            pltpu.VMEM((C, N), jnp.float32),
            pltpu.VMEM((NBUF, TB, C, N), jnp.float32),
            pltpu.VMEM((C, N), jnp.float32),
            pltpu.SemaphoreType.DMA((NBUF,)),
            pltpu.SemaphoreType.DMA,
        ],
    )(xt, tt)
    return out[0, 0]


# speedup vs baseline: 17.1828x; 1.0010x over previous
import functools

import jax
import jax.numpy as jnp
from jax.experimental import pallas as pl
from jax.experimental.pallas import tpu as pltpu

DESIRED = 5.0
UNDESIRED = 1.0


def _body(x_hbm, t_hbm, o_ref, acc_ref, buf_ref, tbuf_ref, sem, tsem,
          *, K, TB, NBUF, scale):
    def start(j):
        slot = jax.lax.rem(j, NBUF)
        pltpu.make_async_copy(
            x_hbm.at[pl.ds(j * TB, TB)],
            buf_ref.at[slot],
            sem.at[slot],
        ).start()

    def wait(j):
        slot = jax.lax.rem(j, NBUF)
        pltpu.make_async_copy(
            x_hbm.at[pl.ds(j * TB, TB)],
            buf_ref.at[slot],
            sem.at[slot],
        ).wait()

    tcopy = pltpu.make_async_copy(t_hbm, tbuf_ref, tsem)
    tcopy.start()
    for j in range(min(NBUF, K)):
        start(j)

    def step(j, carry):
        wait(j)

        slot = jax.lax.rem(j, NBUF)
        s = jnp.sum(buf_ref[slot], axis=0)

        @pl.when(j == 0)
        def _():
            acc_ref[...] = s

        @pl.when(j > 0)
        def _():
            acc_ref[...] += s

        @pl.when(j + NBUF < K)
        def _():
            start(j + NBUF)

        return carry

    jax.lax.fori_loop(0, K, step, 0, unroll=2)

    tcopy.wait()
    t = tbuf_ref[...]
    oc = acc_ref[...]
    oc = jnp.where((t == DESIRED) & (oc > DESIRED), DESIRED, oc)
    oc = jnp.where((t == UNDESIRED) & (oc < UNDESIRED), UNDESIRED, oc)
    d = oc - t
    o_ref[0, 0] = jnp.sum(d * d) * scale


def kernel(output, target):
    T, N, C = output.shape
    TB = 4
    NBUF = 8
    assert T % TB == 0
    K = T // TB
    scale = 0.5 / T

    xt = jnp.transpose(output, (0, 2, 1))
    tt = target.T

    out = pl.pallas_call(
        functools.partial(_body, K=K, TB=TB, NBUF=NBUF, scale=scale),
        in_specs=[
            pl.BlockSpec(memory_space=pl.ANY),
            pl.BlockSpec(memory_space=pl.ANY),
        ],
        out_specs=pl.BlockSpec(memory_space=pltpu.SMEM),
        out_shape=jax.ShapeDtypeStruct((1, 1), jnp.float32),
        scratch_shapes=[
            pltpu.VMEM((C, N), jnp.float32),
            pltpu.VMEM((NBUF, TB, C, N), jnp.float32),
            pltpu.VMEM((C, N), jnp.float32),
            pltpu.SemaphoreType.DMA((NBUF,)),
            pltpu.SemaphoreType.DMA,
        ],
    )(xt, tt)
    return out[0, 0]
